# baseline (speedup 1.0000x reference)
import functools

import jax
import jax.numpy as jnp
from jax import lax
from jax.experimental import pallas as pl
from jax.experimental.pallas import tpu as pltpu
from jax.experimental.pallas import tpu_sc as plsc

NC = 2
NS = 16
CHUNK = 80


def _sc_aggregate(y, ei, n_chunks):
  N, F = y.shape
  EPW = n_chunks * CHUNK
  NP = ((N + NS * 8 - 1) // (NS * 8)) * NS * 8
  RPT = NP // NS
  ZR = RPT // 8
  assert RPT % 8 == 0 and n_chunks >= 5 and (n_chunks - 5) % 4 == 0
  assert CHUNK % 16 == 0
  assert F % 32 == 0 and N % 16 == 0

  mesh = plsc.VectorSubcoreMesh(core_axis_name="c", subcore_axis_name="s")

  def body(y_h, ei_h, acc_o, cnt_o,
           src_v, dst_v, rows_v, zf_v, ones_v, zc_v,
           sg0, sg1, sg2, sg3, sg4, sg5, csem, acc_sh, cnt_sh):
    c = lax.axis_index("c")
    s = lax.axis_index("s")

    base = (c * NS + s) * EPW
    pltpu.sync_copy(ei_h.at[0, pl.ds(base, EPW)], src_v)
    pltpu.sync_copy(ei_h.at[1, pl.ds(base, EPW)], dst_v)

    zeros16 = jnp.zeros((16,), jnp.float32)
    zeros32b = jnp.zeros((32,), jnp.bfloat16)
    ones16 = jnp.ones((16,), jnp.float32)

    def fill_zf(r, carry):
      def inner(k, carry2):
        zf_v[r, pl.ds(k * 32, 32)] = zeros32b
        return carry2
      return lax.fori_loop(0, F // 32, inner, carry)
    lax.fori_loop(0, ZR, fill_zf, 0)

    def fill_ones(r, carry):
      ones_v[r, :] = ones16
      return carry
    lax.fori_loop(0, CHUNK, fill_ones, 0)

    def fill_zc(r, carry):
      zc_v[r, :] = zeros16
      return carry
    lax.fori_loop(0, RPT, fill_zc, 0)

    row0 = s * RPT
    for t in range(RPT // ZR):
      pltpu.sync_copy(zf_v, acc_sh.at[pl.ds(row0 + t * ZR, ZR)])
    pltpu.sync_copy(zc_v, cnt_sh.at[pl.ds(row0, RPT)])
    plsc.subcore_barrier()

    gsems = (sg0, sg1, sg2, sg3, sg4, sg5)

    def issue_g(j, b):
      pltpu.async_copy(y_h.at[src_v.at[pl.ds(j * CHUNK, CHUNK)]],
                       rows_v.at[b], gsems[b])

    def wait_g(j, b):
      pltpu.make_async_copy(y_h.at[src_v.at[pl.ds(j * CHUNK, CHUNK)]],
                            rows_v.at[b], gsems[b]).wait()

    RING = 6

    def step(j, b, do_issue_g):
      wait_g(j, b)
      dsts = dst_v.at[pl.ds(j * CHUNK, CHUNK)]
      pltpu.async_copy(ones_v, cnt_sh.at[dsts], csem, add=True)
      pltpu.sync_copy(rows_v.at[b], acc_sh.at[dsts], add=True)
      pltpu.make_async_copy(ones_v, cnt_sh.at[dsts], csem).wait()
      if do_issue_g:
        issue_g(j + RING - 1, (b + RING - 1) % RING)

    for j in range(RING - 1):
      issue_g(j, j)

    def loop_ring(j4, carry):
      j0 = RING * j4
      for i in range(RING):
        step(j0 + i, i, True)
      return carry
    nfull = (n_chunks - (RING - 1)) // RING
    lax.fori_loop(0, nfull, loop_ring, 0)

    for j in range(RING * nfull, n_chunks):
      step(j, j % RING, j + RING - 1 < n_chunks)
    plsc.subcore_barrier()

    pltpu.sync_copy(acc_sh.at[pl.ds(row0, RPT)], acc_o.at[c, pl.ds(row0, RPT)])
    pltpu.sync_copy(cnt_sh.at[pl.ds(row0, RPT)], cnt_o.at[c, pl.ds(row0, RPT)])

  kfn = pl.kernel(
      body,
      out_type=(jax.ShapeDtypeStruct((NC, NP, F), jnp.bfloat16),
                jax.ShapeDtypeStruct((NC, NP, 16), jnp.float32)),
      mesh=mesh,
      compiler_params=pltpu.CompilerParams(use_tc_tiling_on_sc=False,
                                           needs_layout_passes=False),
      scratch_types=[
          pltpu.VMEM((EPW,), jnp.int32),
          pltpu.VMEM((EPW,), jnp.int32),
          pltpu.VMEM((6, CHUNK, F), jnp.bfloat16),
          pltpu.VMEM((ZR, F), jnp.bfloat16),
          pltpu.VMEM((CHUNK, 16), jnp.float32),
          pltpu.VMEM((RPT, 16), jnp.float32),
          pltpu.SemaphoreType.DMA,
          pltpu.SemaphoreType.DMA,
          pltpu.SemaphoreType.DMA,
          pltpu.SemaphoreType.DMA,
          pltpu.SemaphoreType.DMA,
          pltpu.SemaphoreType.DMA,
          pltpu.SemaphoreType.DMA,
          pltpu.VMEM_SHARED((NP, F), jnp.bfloat16),
          pltpu.VMEM_SHARED((NP, 16), jnp.float32),
      ],
  )
  return kfn(y, ei)


def _t1_body(acc, cnt, x, wl1, bl1, wr1, wl2, wr2, bl2, h_o, y2_o, hr2_o):
  a = acc[0].astype(jnp.float32) + acc[1].astype(jnp.float32)
  c = cnt[0, :, :1] + cnt[1, :, :1]
  agg = a / jnp.maximum(c, 1.0)
  h = jnp.dot(agg, wl1[...], preferred_element_type=jnp.float32)
  h = h + bl1[...]
  h = h + jnp.dot(x[...], wr1[...], preferred_element_type=jnp.float32)
  h = jnp.maximum(h, 0.0)
  h_o[...] = h
  y2 = jnp.dot(h, wl2[...], preferred_element_type=jnp.float32)
  y2_o[...] = y2.astype(jnp.bfloat16)
  hr2_o[...] = jnp.dot(h, wr2[...], preferred_element_type=jnp.float32) + bl2[...]


def _t2_body(acc, cnt, hr2, out_o):
  a = acc[0].astype(jnp.float32) + acc[1].astype(jnp.float32)
  c = cnt[0, :, :1] + cnt[1, :, :1]
  out_o[...] = a / jnp.maximum(c, 1.0) + hr2[...]


def kernel(x, edge_index0, edge_index1, W_l1, b_l1, W_r1, W_l2, b_l2, W_r2):
  N, FIN = x.shape
  H = W_l1.shape[1]
  C = W_l2.shape[1]
  E = edge_index0.shape[1]
  NW = NC * NS
  EPW = E // NW
  n_chunks = EPW // CHUNK

  ei0 = edge_index0.astype(jnp.int32)
  ei1 = edge_index1.astype(jnp.int32)

  x_bf = x.astype(jnp.bfloat16)
  acc0, cnt0 = _sc_aggregate(x_bf, ei0, n_chunks)

  BLK = 1000
  grid = (N // BLK,)
  h, y2, hr2 = pl.pallas_call(
      _t1_body,
      grid=grid,
      in_specs=[
          pl.BlockSpec((NC, BLK, H), lambda i: (0, i, 0)),
          pl.BlockSpec((NC, BLK, 16), lambda i: (0, i, 0)),
          pl.BlockSpec((BLK, FIN), lambda i: (i, 0)),
          pl.BlockSpec((FIN, H), lambda i: (0, 0)),
          pl.BlockSpec((1, H), lambda i: (0, 0)),
          pl.BlockSpec((FIN, H), lambda i: (0, 0)),
          pl.BlockSpec((H, C), lambda i: (0, 0)),
          pl.BlockSpec((H, C), lambda i: (0, 0)),
          pl.BlockSpec((1, C), lambda i: (0, 0)),
      ],
      out_specs=[
          pl.BlockSpec((BLK, H), lambda i: (i, 0)),
          pl.BlockSpec((BLK, C), lambda i: (i, 0)),
          pl.BlockSpec((BLK, C), lambda i: (i, 0)),
      ],
      out_shape=[
          jax.ShapeDtypeStruct((N, H), jnp.float32),
          jax.ShapeDtypeStruct((N, C), jnp.bfloat16),
          jax.ShapeDtypeStruct((N, C), jnp.float32),
      ],
  )(acc0, cnt0, x, W_l1, b_l1.reshape(1, H), W_r1, W_l2, W_r2,
    b_l2.reshape(1, C))

  acc1, cnt1 = _sc_aggregate(y2, ei1, n_chunks)

  out = pl.pallas_call(
      _t2_body,
      grid=grid,
      in_specs=[
          pl.BlockSpec((NC, BLK, C), lambda i: (0, i, 0)),
          pl.BlockSpec((NC, BLK, 16), lambda i: (0, i, 0)),
          pl.BlockSpec((BLK, C), lambda i: (i, 0)),
      ],
      out_specs=pl.BlockSpec((BLK, C), lambda i: (i, 0)),
      out_shape=jax.ShapeDtypeStruct((N, C), jnp.float32),
  )(acc1, cnt1, hr2)

  return out


# speedup vs baseline: 1.0633x; 1.0633x over previous
import functools

import jax
import jax.numpy as jnp
from jax import lax
from jax.experimental import pallas as pl
from jax.experimental.pallas import tpu as pltpu
from jax.experimental.pallas import tpu_sc as plsc

NC = 2
NS = 16
CHUNK = 80


def _sc_aggregate(y, ei, n_chunks):
  N, F = y.shape
  EPW = n_chunks * CHUNK
  NP = ((N + NS * 8 - 1) // (NS * 8)) * NS * 8
  RPT = NP // NS
  ZR = RPT // 8
  assert RPT % 8 == 0 and n_chunks >= 5 and (n_chunks - 5) % 4 == 0
  assert CHUNK % 16 == 0
  assert F % 32 == 0 and N % 16 == 0

  mesh = plsc.VectorSubcoreMesh(core_axis_name="c", subcore_axis_name="s")

  def body(y_h, ei_h, acc_o, cnt_o,
           src_v, dst_v, rows_v, zf_v, cntl_v,
           sg0, sg1, sg2, sg3, sg4, sg5, sg6, sg7, acc_sh):
    c = lax.axis_index("c")
    s = lax.axis_index("s")

    base = (c * NS + s) * EPW
    pltpu.sync_copy(ei_h.at[0, pl.ds(base, EPW)], src_v)
    pltpu.sync_copy(ei_h.at[1, pl.ds(base, EPW)], dst_v)

    zeros16 = jnp.zeros((16,), jnp.float32)
    zeros32b = jnp.zeros((32,), jnp.bfloat16)
    ones16 = jnp.ones((16,), jnp.float32)

    def fill_zf(r, carry):
      def inner(k, carry2):
        zf_v[r, pl.ds(k * 32, 32)] = zeros32b
        return carry2
      return lax.fori_loop(0, F // 32, inner, carry)
    lax.fori_loop(0, ZR, fill_zf, 0)

    def fill_cnt(r, carry):
      cntl_v[pl.ds(r * 16, 16)] = zeros16
      return carry
    lax.fori_loop(0, N // 16, fill_cnt, 0)

    row0 = s * RPT
    for t in range(RPT // ZR):
      pltpu.sync_copy(zf_v, acc_sh.at[pl.ds(row0 + t * ZR, ZR)])
    plsc.subcore_barrier()

    gsems = (sg0, sg1, sg2, sg3, sg4, sg5, sg6, sg7)

    def issue_g(j, b):
      pltpu.async_copy(y_h.at[src_v.at[pl.ds(j * CHUNK, CHUNK)]],
                       rows_v.at[b], gsems[b])

    def wait_g(j, b):
      pltpu.make_async_copy(y_h.at[src_v.at[pl.ds(j * CHUNK, CHUNK)]],
                            rows_v.at[b], gsems[b]).wait()

    RING = 8

    def step(j, b, do_issue_g):
      wait_g(j, b)
      pltpu.sync_copy(rows_v.at[b],
                      acc_sh.at[dst_v.at[pl.ds(j * CHUNK, CHUNK)]],
                      add=True)
      for k in range(CHUNK // 16):
        idx = dst_v[pl.ds(j * CHUNK + k * 16, 16)]
        plsc.addupdate_scatter(cntl_v, [idx], ones16)
      if do_issue_g:
        issue_g(j + RING - 1, (b + RING - 1) % RING)

    for j in range(RING - 1):
      issue_g(j, j)

    def loop_ring(j4, carry):
      j0 = RING * j4
      for i in range(RING):
        step(j0 + i, i, True)
      return carry
    nfull = (n_chunks - (RING - 1)) // RING
    lax.fori_loop(0, nfull, loop_ring, 0)

    for j in range(RING * nfull, n_chunks):
      step(j, j % RING, j + RING - 1 < n_chunks)
    plsc.subcore_barrier()

    pltpu.sync_copy(acc_sh.at[pl.ds(row0, RPT)], acc_o.at[c, pl.ds(row0, RPT)])
    pltpu.sync_copy(cntl_v, cnt_o.at[c, s])

  kfn = pl.kernel(
      body,
      out_type=(jax.ShapeDtypeStruct((NC, NP, F), jnp.bfloat16),
                jax.ShapeDtypeStruct((NC, NS, N), jnp.float32)),
      mesh=mesh,
      compiler_params=pltpu.CompilerParams(use_tc_tiling_on_sc=False,
                                           needs_layout_passes=False),
      scratch_types=[
          pltpu.VMEM((EPW,), jnp.int32),
          pltpu.VMEM((EPW,), jnp.int32),
          pltpu.VMEM((8, CHUNK, F), jnp.bfloat16),
          pltpu.VMEM((ZR, F), jnp.bfloat16),
          pltpu.VMEM((N,), jnp.float32),
          pltpu.SemaphoreType.DMA,
          pltpu.SemaphoreType.DMA,
          pltpu.SemaphoreType.DMA,
          pltpu.SemaphoreType.DMA,
          pltpu.SemaphoreType.DMA,
          pltpu.SemaphoreType.DMA,
          pltpu.SemaphoreType.DMA,
          pltpu.SemaphoreType.DMA,
          pltpu.VMEM_SHARED((NP, F), jnp.bfloat16),
      ],
  )
  return kfn(y, ei)


def _t1_body(acc, cnt, x, wl1, bl1, wr1, wl2, wr2, bl2, h_o, y2_o, hr2_o):
  a = acc[0].astype(jnp.float32) + acc[1].astype(jnp.float32)
  c = jnp.sum(cnt[...], axis=1, keepdims=True)
  agg = a / jnp.maximum(c, 1.0)
  h = jnp.dot(agg, wl1[...], preferred_element_type=jnp.float32)
  h = h + bl1[...]
  h = h + jnp.dot(x[...], wr1[...], preferred_element_type=jnp.float32)
  h = jnp.maximum(h, 0.0)
  h_o[...] = h
  y2 = jnp.dot(h, wl2[...], preferred_element_type=jnp.float32)
  y2_o[...] = y2.astype(jnp.bfloat16)
  hr2_o[...] = jnp.dot(h, wr2[...], preferred_element_type=jnp.float32) + bl2[...]


def _t2_body(acc, cnt, hr2, out_o):
  a = acc[0].astype(jnp.float32) + acc[1].astype(jnp.float32)
  c = jnp.sum(cnt[...], axis=1, keepdims=True)
  out_o[...] = a / jnp.maximum(c, 1.0) + hr2[...]


def kernel(x, edge_index0, edge_index1, W_l1, b_l1, W_r1, W_l2, b_l2, W_r2):
  N, FIN = x.shape
  H = W_l1.shape[1]
  C = W_l2.shape[1]
  E = edge_index0.shape[1]
  NW = NC * NS
  EPW = E // NW
  n_chunks = EPW // CHUNK

  ei0 = edge_index0.astype(jnp.int32)
  ei1 = edge_index1.astype(jnp.int32)

  x_bf = x.astype(jnp.bfloat16)
  acc0, cnt0 = _sc_aggregate(x_bf, ei0, n_chunks)
  cnt0_t = jnp.transpose(cnt0, (2, 0, 1)).reshape(N, NW)

  BLK = 1000
  grid = (N // BLK,)
  h, y2, hr2 = pl.pallas_call(
      _t1_body,
      grid=grid,
      in_specs=[
          pl.BlockSpec((NC, BLK, H), lambda i: (0, i, 0)),
          pl.BlockSpec((BLK, NW), lambda i: (i, 0)),
          pl.BlockSpec((BLK, FIN), lambda i: (i, 0)),
          pl.BlockSpec((FIN, H), lambda i: (0, 0)),
          pl.BlockSpec((1, H), lambda i: (0, 0)),
          pl.BlockSpec((FIN, H), lambda i: (0, 0)),
          pl.BlockSpec((H, C), lambda i: (0, 0)),
          pl.BlockSpec((H, C), lambda i: (0, 0)),
          pl.BlockSpec((1, C), lambda i: (0, 0)),
      ],
      out_specs=[
          pl.BlockSpec((BLK, H), lambda i: (i, 0)),
          pl.BlockSpec((BLK, C), lambda i: (i, 0)),
          pl.BlockSpec((BLK, C), lambda i: (i, 0)),
      ],
      out_shape=[
          jax.ShapeDtypeStruct((N, H), jnp.float32),
          jax.ShapeDtypeStruct((N, C), jnp.bfloat16),
          jax.ShapeDtypeStruct((N, C), jnp.float32),
      ],
  )(acc0, cnt0_t, x, W_l1, b_l1.reshape(1, H), W_r1, W_l2, W_r2,
    b_l2.reshape(1, C))

  acc1, cnt1 = _sc_aggregate(y2, ei1, n_chunks)
  cnt1_t = jnp.transpose(cnt1, (2, 0, 1)).reshape(N, NW)

  out = pl.pallas_call(
      _t2_body,
      grid=grid,
      in_specs=[
          pl.BlockSpec((NC, BLK, C), lambda i: (0, i, 0)),
          pl.BlockSpec((BLK, NW), lambda i: (i, 0)),
          pl.BlockSpec((BLK, C), lambda i: (i, 0)),
      ],
      out_specs=pl.BlockSpec((BLK, C), lambda i: (i, 0)),
      out_shape=jax.ShapeDtypeStruct((N, C), jnp.float32),
  )(acc1, cnt1_t, hr2)

  return out


# speedup vs baseline: 1.0920x; 1.0269x over previous
import functools

import jax
import jax.numpy as jnp
from jax import lax
from jax.experimental import pallas as pl
from jax.experimental.pallas import tpu as pltpu
from jax.experimental.pallas import tpu_sc as plsc

NC = 2
NS = 16
CHUNK = 80


def _sc_aggregate(y, ei, n_chunks):
  N, F = y.shape
  EPW = n_chunks * CHUNK
  NP = ((N + NS * 8 - 1) // (NS * 8)) * NS * 8
  RPT = NP // NS
  ZR = RPT // 8
  assert RPT % 8 == 0 and n_chunks >= 5 and (n_chunks - 5) % 4 == 0
  assert CHUNK % 16 == 0
  assert F % 32 == 0 and N % 16 == 0

  mesh = plsc.VectorSubcoreMesh(core_axis_name="c", subcore_axis_name="s")

  def body(y_h, ei_h, acc_o, cnt_o,
           src_v, dst_v, rows_v, zf_v, cntl_v,
           sg0, sg1, sg2, sg3, sg4, sg5, sg6, sg7, acc_sh):
    c = lax.axis_index("c")
    s = lax.axis_index("s")

    base = (c * NS + s) * EPW
    pltpu.sync_copy(ei_h.at[0, pl.ds(base, EPW)], src_v)
    pltpu.sync_copy(ei_h.at[1, pl.ds(base, EPW)], dst_v)

    zeros16 = jnp.zeros((16,), jnp.float32)
    zeros32b = jnp.zeros((32,), jnp.bfloat16)
    ones16 = jnp.ones((16,), jnp.float32)

    def fill_zf(r, carry):
      def inner(k, carry2):
        zf_v[r, pl.ds(k * 32, 32)] = zeros32b
        return carry2
      return lax.fori_loop(0, F // 32, inner, carry)
    lax.fori_loop(0, ZR, fill_zf, 0)

    def fill_cnt(r, carry):
      cntl_v[pl.ds(r * 16, 16)] = zeros16
      return carry
    lax.fori_loop(0, N // 16, fill_cnt, 0)

    row0 = s * RPT
    for t in range(RPT // ZR):
      pltpu.sync_copy(zf_v, acc_sh.at[pl.ds(row0 + t * ZR, ZR)])
    plsc.subcore_barrier()

    gsems = (sg0, sg1, sg2, sg3, sg4, sg5, sg6, sg7)

    def issue_g(j, b):
      pltpu.async_copy(y_h.at[src_v.at[pl.ds(j * CHUNK, CHUNK)]],
                       rows_v.at[b], gsems[b])

    def wait_g(j, b):
      pltpu.make_async_copy(y_h.at[src_v.at[pl.ds(j * CHUNK, CHUNK)]],
                            rows_v.at[b], gsems[b]).wait()

    RING = 8

    def step(j, b, do_issue_g):
      wait_g(j, b)
      pltpu.sync_copy(rows_v.at[b],
                      acc_sh.at[dst_v.at[pl.ds(j * CHUNK, CHUNK)]],
                      add=True)
      for k in range(CHUNK // 16):
        idx = dst_v[pl.ds(j * CHUNK + k * 16, 16)]
        plsc.addupdate_scatter(cntl_v, [idx], ones16)
      if do_issue_g:
        issue_g(j + RING - 1, (b + RING - 1) % RING)

    for j in range(RING - 1):
      issue_g(j, j)

    def loop_ring(j4, carry):
      j0 = RING * j4
      for i in range(RING):
        step(j0 + i, i, True)
      return carry
    nfull = (n_chunks - (RING - 1)) // RING
    lax.fori_loop(0, nfull, loop_ring, 0)

    for j in range(RING * nfull, n_chunks):
      step(j, j % RING, j + RING - 1 < n_chunks)
    plsc.subcore_barrier()

    pltpu.sync_copy(acc_sh.at[pl.ds(row0, RPT)], acc_o.at[c, pl.ds(row0, RPT)])
    pltpu.sync_copy(cntl_v, cnt_o.at[c, s])

  kfn = pl.kernel(
      body,
      out_type=(jax.ShapeDtypeStruct((NC, NP, F), jnp.bfloat16),
                jax.ShapeDtypeStruct((NC, NS, N), jnp.float32)),
      mesh=mesh,
      compiler_params=pltpu.CompilerParams(use_tc_tiling_on_sc=False,
                                           needs_layout_passes=False),
      scratch_types=[
          pltpu.VMEM((EPW,), jnp.int32),
          pltpu.VMEM((EPW,), jnp.int32),
          pltpu.VMEM((8, CHUNK, F), jnp.bfloat16),
          pltpu.VMEM((ZR, F), jnp.bfloat16),
          pltpu.VMEM((N,), jnp.float32),
          pltpu.SemaphoreType.DMA,
          pltpu.SemaphoreType.DMA,
          pltpu.SemaphoreType.DMA,
          pltpu.SemaphoreType.DMA,
          pltpu.SemaphoreType.DMA,
          pltpu.SemaphoreType.DMA,
          pltpu.SemaphoreType.DMA,
          pltpu.SemaphoreType.DMA,
          pltpu.VMEM_SHARED((NP, F), jnp.bfloat16),
      ],
  )
  return kfn(y, ei)


def _t1_body(acc, cnt, x, wl1, bl1, wr1, wl2, wr2, bl2, h_o, y2_o, hr2_o):
  a = acc[0].astype(jnp.float32) + acc[1].astype(jnp.float32)
  c = jnp.sum(cnt[...], axis=1, keepdims=True)
  agg = a / jnp.maximum(c, 1.0)
  h = jnp.dot(agg, wl1[...], preferred_element_type=jnp.float32)
  h = h + bl1[...]
  h = h + jnp.dot(x[...], wr1[...], preferred_element_type=jnp.float32)
  h = jnp.maximum(h, 0.0)
  h_o[...] = h
  y2 = jnp.dot(h, wl2[...], preferred_element_type=jnp.float32)
  y2_o[...] = y2.astype(jnp.bfloat16)
  hr2_o[...] = jnp.dot(h, wr2[...], preferred_element_type=jnp.float32) + bl2[...]


def _t2_body(acc, cnt, hr2, out_o):
  a = acc[0].astype(jnp.float32) + acc[1].astype(jnp.float32)
  c = jnp.sum(cnt[...], axis=1, keepdims=True)
  out_o[...] = a / jnp.maximum(c, 1.0) + hr2[...]


def kernel(x, edge_index0, edge_index1, W_l1, b_l1, W_r1, W_l2, b_l2, W_r2):
  N, FIN = x.shape
  H = W_l1.shape[1]
  C = W_l2.shape[1]
  E = edge_index0.shape[1]
  NW = NC * NS
  EPW = E // NW
  n_chunks = EPW // CHUNK

  ei0 = edge_index0.astype(jnp.int32)
  ei1 = edge_index1.astype(jnp.int32)

  x_bf = x.astype(jnp.bfloat16)
  acc0, cnt0 = _sc_aggregate(x_bf, ei0, n_chunks)
  cnt0_t = jnp.transpose(cnt0, (2, 0, 1)).reshape(N, NW)

  BLK = 2000
  grid = (N // BLK,)
  h, y2, hr2 = pl.pallas_call(
      _t1_body,
      grid=grid,
      in_specs=[
          pl.BlockSpec((NC, BLK, H), lambda i: (0, i, 0)),
          pl.BlockSpec((BLK, NW), lambda i: (i, 0)),
          pl.BlockSpec((BLK, FIN), lambda i: (i, 0)),
          pl.BlockSpec((FIN, H), lambda i: (0, 0)),
          pl.BlockSpec((1, H), lambda i: (0, 0)),
          pl.BlockSpec((FIN, H), lambda i: (0, 0)),
          pl.BlockSpec((H, C), lambda i: (0, 0)),
          pl.BlockSpec((H, C), lambda i: (0, 0)),
          pl.BlockSpec((1, C), lambda i: (0, 0)),
      ],
      out_specs=[
          pl.BlockSpec((BLK, H), lambda i: (i, 0)),
          pl.BlockSpec((BLK, C), lambda i: (i, 0)),
          pl.BlockSpec((BLK, C), lambda i: (i, 0)),
      ],
      out_shape=[
          jax.ShapeDtypeStruct((N, H), jnp.float32),
          jax.ShapeDtypeStruct((N, C), jnp.bfloat16),
          jax.ShapeDtypeStruct((N, C), jnp.float32),
      ],
  )(acc0, cnt0_t, x, W_l1, b_l1.reshape(1, H), W_r1, W_l2, W_r2,
    b_l2.reshape(1, C))

  acc1, cnt1 = _sc_aggregate(y2, ei1, n_chunks)
  cnt1_t = jnp.transpose(cnt1, (2, 0, 1)).reshape(N, NW)

  out = pl.pallas_call(
      _t2_body,
      grid=grid,
      in_specs=[
          pl.BlockSpec((NC, BLK, C), lambda i: (0, i, 0)),
          pl.BlockSpec((BLK, NW), lambda i: (i, 0)),
          pl.BlockSpec((BLK, C), lambda i: (i, 0)),
      ],
      out_specs=pl.BlockSpec((BLK, C), lambda i: (i, 0)),
      out_shape=jax.ShapeDtypeStruct((N, C), jnp.float32),
  )(acc1, cnt1_t, hr2)

  return out


# speedup vs baseline: 1.0963x; 1.0039x over previous
import functools

import jax
import jax.numpy as jnp
from jax import lax
from jax.experimental import pallas as pl
from jax.experimental.pallas import tpu as pltpu
from jax.experimental.pallas import tpu_sc as plsc

NC = 2
NS = 16
CHUNK = 80


def _sc_aggregate(y, ei, n_chunks):
  N, F = y.shape
  EPW = n_chunks * CHUNK
  NP = ((N + NS * 8 - 1) // (NS * 8)) * NS * 8
  RPT = NP // NS
  ZR = RPT // 8
  assert RPT % 8 == 0 and n_chunks >= 5 and (n_chunks - 5) % 4 == 0
  assert CHUNK % 16 == 0
  assert F % 32 == 0 and N % 16 == 0

  mesh = plsc.VectorSubcoreMesh(core_axis_name="c", subcore_axis_name="s")

  def body(y_h, ei_h, acc_o, cnt_o,
           src_v, dst_v, rows_v, zf_v, cntl_v,
           sg0, sg1, sg2, sg3, sg4, sg5, sg6, sg7, acc_sh):
    c = lax.axis_index("c")
    s = lax.axis_index("s")

    base = (c * NS + s) * EPW
    pltpu.sync_copy(ei_h.at[0, pl.ds(base, EPW)], src_v)
    pltpu.sync_copy(ei_h.at[1, pl.ds(base, EPW)], dst_v)

    zeros16 = jnp.zeros((16,), jnp.float32)
    zeros32b = jnp.zeros((32,), jnp.bfloat16)
    ones16 = jnp.ones((16,), jnp.float32)

    def fill_zf(r, carry):
      def inner(k, carry2):
        zf_v[r, pl.ds(k * 32, 32)] = zeros32b
        return carry2
      return lax.fori_loop(0, F // 32, inner, carry)
    lax.fori_loop(0, ZR, fill_zf, 0)

    def fill_cnt(r, carry):
      cntl_v[pl.ds(r * 16, 16)] = zeros16
      return carry
    lax.fori_loop(0, N // 16, fill_cnt, 0)

    row0 = s * RPT
    for t in range(RPT // ZR):
      pltpu.sync_copy(zf_v, acc_sh.at[pl.ds(row0 + t * ZR, ZR)])
    plsc.subcore_barrier()

    gsems = (sg0, sg1, sg2, sg3, sg4, sg5, sg6, sg7)

    def issue_g(j, b):
      pltpu.async_copy(y_h.at[src_v.at[pl.ds(j * CHUNK, CHUNK)]],
                       rows_v.at[b], gsems[b])

    def wait_g(j, b):
      pltpu.make_async_copy(y_h.at[src_v.at[pl.ds(j * CHUNK, CHUNK)]],
                            rows_v.at[b], gsems[b]).wait()

    RING = 8

    def step(j, b, do_issue_g):
      wait_g(j, b)
      pltpu.sync_copy(rows_v.at[b],
                      acc_sh.at[dst_v.at[pl.ds(j * CHUNK, CHUNK)]],
                      add=True)
      for k in range(CHUNK // 16):
        idx = dst_v[pl.ds(j * CHUNK + k * 16, 16)]
        plsc.addupdate_scatter(cntl_v, [idx], ones16)
      if do_issue_g:
        issue_g(j + RING - 1, (b + RING - 1) % RING)

    for j in range(RING - 1):
      issue_g(j, j)

    def loop_ring(j4, carry):
      j0 = RING * j4
      for i in range(RING):
        step(j0 + i, i, True)
      return carry
    nfull = (n_chunks - (RING - 1)) // RING
    lax.fori_loop(0, nfull, loop_ring, 0)

    for j in range(RING * nfull, n_chunks):
      step(j, j % RING, j + RING - 1 < n_chunks)
    plsc.subcore_barrier()

    pltpu.sync_copy(acc_sh.at[pl.ds(row0, RPT)], acc_o.at[c, pl.ds(row0, RPT)])
    pltpu.sync_copy(cntl_v, cnt_o.at[c, s])

  kfn = pl.kernel(
      body,
      out_type=(jax.ShapeDtypeStruct((NC, NP, F), jnp.bfloat16),
                jax.ShapeDtypeStruct((NC, NS, N), jnp.float32)),
      mesh=mesh,
      compiler_params=pltpu.CompilerParams(use_tc_tiling_on_sc=False,
                                           needs_layout_passes=False),
      scratch_types=[
          pltpu.VMEM((EPW,), jnp.int32),
          pltpu.VMEM((EPW,), jnp.int32),
          pltpu.VMEM((8, CHUNK, F), jnp.bfloat16),
          pltpu.VMEM((ZR, F), jnp.bfloat16),
          pltpu.VMEM((N,), jnp.float32),
          pltpu.SemaphoreType.DMA,
          pltpu.SemaphoreType.DMA,
          pltpu.SemaphoreType.DMA,
          pltpu.SemaphoreType.DMA,
          pltpu.SemaphoreType.DMA,
          pltpu.SemaphoreType.DMA,
          pltpu.SemaphoreType.DMA,
          pltpu.SemaphoreType.DMA,
          pltpu.VMEM_SHARED((NP, F), jnp.bfloat16),
      ],
  )
  return kfn(y, ei)


def _t1_body(acc, cnt, x, wl1, bl1, wr1, wl2, wr2, bl2, h_o, y2_o, hr2_o):
  a = acc[0].astype(jnp.float32) + acc[1].astype(jnp.float32)
  c = jnp.sum(cnt[...], axis=1, keepdims=True)
  agg = a / jnp.maximum(c, 1.0)
  h = jnp.dot(agg, wl1[...], preferred_element_type=jnp.float32)
  h = h + bl1[...]
  h = h + jnp.dot(x[...], wr1[...], preferred_element_type=jnp.float32)
  h = jnp.maximum(h, 0.0)
  h_o[...] = h
  y2 = jnp.dot(h, wl2[...], preferred_element_type=jnp.float32)
  y2_o[...] = y2.astype(jnp.bfloat16)
  hr2_o[...] = jnp.dot(h, wr2[...], preferred_element_type=jnp.float32) + bl2[...]


def _t2_body(acc, cnt, hr2, out_o):
  a = acc[0].astype(jnp.float32) + acc[1].astype(jnp.float32)
  c = jnp.sum(cnt[...], axis=1, keepdims=True)
  out_o[...] = a / jnp.maximum(c, 1.0) + hr2[...]


def kernel(x, edge_index0, edge_index1, W_l1, b_l1, W_r1, W_l2, b_l2, W_r2):
  N, FIN = x.shape
  H = W_l1.shape[1]
  C = W_l2.shape[1]
  E = edge_index0.shape[1]
  NW = NC * NS
  EPW = E // NW
  n_chunks = EPW // CHUNK

  ei0 = edge_index0.astype(jnp.int32)
  ei1 = edge_index1.astype(jnp.int32)

  x_bf = x.astype(jnp.bfloat16)
  acc0, cnt0 = _sc_aggregate(x_bf, ei0, n_chunks)
  cnt0_t = jnp.transpose(cnt0, (2, 0, 1)).reshape(N, NW)

  BLK = 5000
  grid = (N // BLK,)
  h, y2, hr2 = pl.pallas_call(
      _t1_body,
      grid=grid,
      in_specs=[
          pl.BlockSpec((NC, BLK, H), lambda i: (0, i, 0)),
          pl.BlockSpec((BLK, NW), lambda i: (i, 0)),
          pl.BlockSpec((BLK, FIN), lambda i: (i, 0)),
          pl.BlockSpec((FIN, H), lambda i: (0, 0)),
          pl.BlockSpec((1, H), lambda i: (0, 0)),
          pl.BlockSpec((FIN, H), lambda i: (0, 0)),
          pl.BlockSpec((H, C), lambda i: (0, 0)),
          pl.BlockSpec((H, C), lambda i: (0, 0)),
          pl.BlockSpec((1, C), lambda i: (0, 0)),
      ],
      out_specs=[
          pl.BlockSpec((BLK, H), lambda i: (i, 0)),
          pl.BlockSpec((BLK, C), lambda i: (i, 0)),
          pl.BlockSpec((BLK, C), lambda i: (i, 0)),
      ],
      out_shape=[
          jax.ShapeDtypeStruct((N, H), jnp.float32),
          jax.ShapeDtypeStruct((N, C), jnp.bfloat16),
          jax.ShapeDtypeStruct((N, C), jnp.float32),
      ],
  )(acc0, cnt0_t, x, W_l1, b_l1.reshape(1, H), W_r1, W_l2, W_r2,
    b_l2.reshape(1, C))

  acc1, cnt1 = _sc_aggregate(y2, ei1, n_chunks)
  cnt1_t = jnp.transpose(cnt1, (2, 0, 1)).reshape(N, NW)

  out = pl.pallas_call(
      _t2_body,
      grid=grid,
      in_specs=[
          pl.BlockSpec((NC, BLK, C), lambda i: (0, i, 0)),
          pl.BlockSpec((BLK, NW), lambda i: (i, 0)),
          pl.BlockSpec((BLK, C), lambda i: (i, 0)),
      ],
      out_specs=pl.BlockSpec((BLK, C), lambda i: (i, 0)),
      out_shape=jax.ShapeDtypeStruct((N, C), jnp.float32),
  )(acc1, cnt1_t, hr2)

  return out


# speedup vs baseline: 1.0963x; 1.0000x over previous
import jax
import jax.numpy as jnp
from jax import lax
from jax.experimental import pallas as pl
from jax.experimental.pallas import tpu as pltpu
from jax.experimental.pallas import tpu_sc as plsc

NC = 2
NS = 16
CHUNK = 80


def _sc_aggregate(y, ei, n_chunks):
  N, F = y.shape
  EPW = n_chunks * CHUNK
  NP = ((N + NS * 8 - 1) // (NS * 8)) * NS * 8
  RPT = NP // NS
  ZR = RPT // 8
  assert RPT % 8 == 0 and n_chunks >= 5 and (n_chunks - 5) % 4 == 0
  assert CHUNK % 16 == 0
  assert F % 32 == 0 and N % 16 == 0

  mesh = plsc.VectorSubcoreMesh(core_axis_name="c", subcore_axis_name="s")

  def body(y_h, ei_h, acc_o, cnt_o,
           src_v, dst_v, rows_v, zf_v, cntl_v,
           sg0, sg1, sg2, sg3, sg4, sg5, sg6, sg7, acc_sh):
    c = lax.axis_index("c")
    s = lax.axis_index("s")

    base = (c * NS + s) * EPW
    pltpu.sync_copy(ei_h.at[0, pl.ds(base, EPW)], src_v)
    pltpu.sync_copy(ei_h.at[1, pl.ds(base, EPW)], dst_v)

    zeros16 = jnp.zeros((16,), jnp.float32)
    zeros32b = jnp.zeros((32,), jnp.bfloat16)
    ones16 = jnp.ones((16,), jnp.float32)

    def fill_zf(r, carry):
      def inner(k, carry2):
        zf_v[r, pl.ds(k * 32, 32)] = zeros32b
        return carry2
      return lax.fori_loop(0, F // 32, inner, carry)
    lax.fori_loop(0, ZR, fill_zf, 0)

    def fill_cnt(r, carry):
      cntl_v[pl.ds(r * 16, 16)] = zeros16
      return carry
    lax.fori_loop(0, N // 16, fill_cnt, 0)

    row0 = s * RPT
    for t in range(RPT // ZR):
      pltpu.sync_copy(zf_v, acc_sh.at[pl.ds(row0 + t * ZR, ZR)])
    plsc.subcore_barrier()

    gsems = (sg0, sg1, sg2, sg3, sg4, sg5, sg6, sg7)

    def issue_g(j, b):
      pltpu.async_copy(y_h.at[src_v.at[pl.ds(j * CHUNK, CHUNK)]],
                       rows_v.at[b], gsems[b])

    def wait_g(j, b):
      pltpu.make_async_copy(y_h.at[src_v.at[pl.ds(j * CHUNK, CHUNK)]],
                            rows_v.at[b], gsems[b]).wait()

    RING = 8

    def step(j, b, do_issue_g):
      wait_g(j, b)
      pltpu.sync_copy(rows_v.at[b],
                      acc_sh.at[dst_v.at[pl.ds(j * CHUNK, CHUNK)]],
                      add=True)
      for k in range(CHUNK // 16):
        idx = dst_v[pl.ds(j * CHUNK + k * 16, 16)]
        plsc.addupdate_scatter(cntl_v, [idx], ones16)
      if do_issue_g:
        issue_g(j + RING - 1, (b + RING - 1) % RING)

    for j in range(RING - 1):
      issue_g(j, j)

    def loop_ring(j4, carry):
      j0 = RING * j4
      for i in range(RING):
        step(j0 + i, i, True)
      return carry
    nfull = (n_chunks - (RING - 1)) // RING
    lax.fori_loop(0, nfull, loop_ring, 0)

    for j in range(RING * nfull, n_chunks):
      step(j, j % RING, j + RING - 1 < n_chunks)
    plsc.subcore_barrier()

    pltpu.sync_copy(acc_sh.at[pl.ds(row0, RPT)], acc_o.at[c, pl.ds(row0, RPT)])
    pltpu.sync_copy(cntl_v, cnt_o.at[c, s])

  kfn = pl.kernel(
      body,
      out_type=(jax.ShapeDtypeStruct((NC, NP, F), jnp.bfloat16),
                jax.ShapeDtypeStruct((NC, NS, N), jnp.float32)),
      mesh=mesh,
      compiler_params=pltpu.CompilerParams(use_tc_tiling_on_sc=False,
                                           needs_layout_passes=False),
      scratch_types=[
          pltpu.VMEM((EPW,), jnp.int32),
          pltpu.VMEM((EPW,), jnp.int32),
          pltpu.VMEM((8, CHUNK, F), jnp.bfloat16),
          pltpu.VMEM((ZR, F), jnp.bfloat16),
          pltpu.VMEM((N,), jnp.float32),
          pltpu.SemaphoreType.DMA,
          pltpu.SemaphoreType.DMA,
          pltpu.SemaphoreType.DMA,
          pltpu.SemaphoreType.DMA,
          pltpu.SemaphoreType.DMA,
          pltpu.SemaphoreType.DMA,
          pltpu.SemaphoreType.DMA,
          pltpu.SemaphoreType.DMA,
          pltpu.VMEM_SHARED((NP, F), jnp.bfloat16),
      ],
  )
  return kfn(y, ei)


def _t1_body(acc, cnt, x, wl1, bl1, wr1, wl2, wr2, bl2, h_o, y2_o, hr2_o):
  a = acc[0].astype(jnp.float32) + acc[1].astype(jnp.float32)
  c = jnp.sum(cnt[...], axis=1, keepdims=True)
  agg = a / jnp.maximum(c, 1.0)
  h = jnp.dot(agg, wl1[...], preferred_element_type=jnp.float32)
  h = h + bl1[...]
  h = h + jnp.dot(x[...], wr1[...], preferred_element_type=jnp.float32)
  h = jnp.maximum(h, 0.0)
  h_o[...] = h
  y2 = jnp.dot(h, wl2[...], preferred_element_type=jnp.float32)
  y2_o[...] = y2.astype(jnp.bfloat16)
  hr2_o[...] = jnp.dot(h, wr2[...], preferred_element_type=jnp.float32) + bl2[...]


def _t2_body(acc, cnt, hr2, out_o):
  a = acc[0].astype(jnp.float32) + acc[1].astype(jnp.float32)
  c = jnp.sum(cnt[...], axis=1, keepdims=True)
  out_o[...] = a / jnp.maximum(c, 1.0) + hr2[...]


def kernel(x, edge_index0, edge_index1, W_l1, b_l1, W_r1, W_l2, b_l2, W_r2):
  N, FIN = x.shape
  H = W_l1.shape[1]
  C = W_l2.shape[1]
  E = edge_index0.shape[1]
  NW = NC * NS
  EPW = E // NW
  n_chunks = EPW // CHUNK

  ei0 = edge_index0.astype(jnp.int32)
  ei1 = edge_index1.astype(jnp.int32)

  x_bf = x.astype(jnp.bfloat16)
  acc0, cnt0 = _sc_aggregate(x_bf, ei0, n_chunks)
  cnt0_t = jnp.transpose(cnt0, (2, 0, 1)).reshape(N, NW)

  BLK = 5000
  grid = (N // BLK,)
  h, y2, hr2 = pl.pallas_call(
      _t1_body,
      grid=grid,
      in_specs=[
          pl.BlockSpec((NC, BLK, H), lambda i: (0, i, 0)),
          pl.BlockSpec((BLK, NW), lambda i: (i, 0)),
          pl.BlockSpec((BLK, FIN), lambda i: (i, 0)),
          pl.BlockSpec((FIN, H), lambda i: (0, 0)),
          pl.BlockSpec((1, H), lambda i: (0, 0)),
          pl.BlockSpec((FIN, H), lambda i: (0, 0)),
          pl.BlockSpec((H, C), lambda i: (0, 0)),
          pl.BlockSpec((H, C), lambda i: (0, 0)),
          pl.BlockSpec((1, C), lambda i: (0, 0)),
      ],
      out_specs=[
          pl.BlockSpec((BLK, H), lambda i: (i, 0)),
          pl.BlockSpec((BLK, C), lambda i: (i, 0)),
          pl.BlockSpec((BLK, C), lambda i: (i, 0)),
      ],
      out_shape=[
          jax.ShapeDtypeStruct((N, H), jnp.float32),
          jax.ShapeDtypeStruct((N, C), jnp.bfloat16),
          jax.ShapeDtypeStruct((N, C), jnp.float32),
      ],
  )(acc0, cnt0_t, x, W_l1, b_l1.reshape(1, H), W_r1, W_l2, W_r2,
    b_l2.reshape(1, C))

  acc1, cnt1 = _sc_aggregate(y2, ei1, n_chunks)
  cnt1_t = jnp.transpose(cnt1, (2, 0, 1)).reshape(N, NW)

  out = pl.pallas_call(
      _t2_body,
      grid=grid,
      in_specs=[
          pl.BlockSpec((NC, BLK, C), lambda i: (0, i, 0)),
          pl.BlockSpec((BLK, NW), lambda i: (i, 0)),
          pl.BlockSpec((BLK, C), lambda i: (i, 0)),
      ],
      out_specs=pl.BlockSpec((BLK, C), lambda i: (i, 0)),
      out_shape=jax.ShapeDtypeStruct((N, C), jnp.float32),
  )(acc1, cnt1_t, hr2)

  return out
